# Initial kernel scaffold; baseline (speedup 1.0000x reference)
import jax
import jax.numpy as jnp
from jax.experimental import pallas as pl


def kernel(x, n_id, edge_index0, edge_index1, edge_index2, res_n_id1, res_n_id2, W1, b1, W2, b2, W3, b3, W4, b4):
    raise NotImplementedError("write your pallas kernel here")


# speedup vs baseline: 5.8495x; 5.8495x over previous
import functools

import jax
import jax.numpy as jnp
from jax import lax
from jax.experimental import pallas as pl
from jax.experimental.pallas import tpu as pltpu
from jax.experimental.pallas import tpu_sc as plsc

N0, N1, N2 = 30000, 8000, 2000
D = 128
NID_PAD = 32768
E0P, E1P, E2P = 483328, 131072, 32768
ACC0, ACC1, ACC2 = 30720, 8192, 2048
NW = 32


def _mesh():
    return plsc.VectorSubcoreMesh(core_axis_name="c", subcore_axis_name="s")


def _sc_gather(x2, nid_p):
    blocks = NID_PAD // (NW * 128)

    @functools.partial(
        pl.kernel, mesh=_mesh(),
        out_type=jax.ShapeDtypeStruct((NID_PAD, D), jnp.float32),
        scratch_types=[
            pltpu.VMEM((128,), jnp.int32),
            pltpu.VMEM((128, D), jnp.float32),
            pltpu.SemaphoreType.DMA,
        ])
    def k(x_h, nid_h, t_h, idx_v, rows_v, sem):
        wid = lax.axis_index("c") * 16 + lax.axis_index("s")

        def body(b, c):
            base = (wid * blocks + b) * 128
            pltpu.sync_copy(nid_h.at[pl.ds(base, 128)], idx_v)
            pltpu.async_copy(x_h.at[idx_v], rows_v, sem).wait()
            pltpu.sync_copy(rows_v, t_h.at[pl.ds(base, 128)])
            return c

        lax.fori_loop(0, blocks, body, 0)

    return k(x2, nid_p)


def _sc_segsum(msg, src, dst, n_acc, width, ep):
    blocks = ep // (NW * 128)
    rpt = n_acc // 16
    chunks = rpt // 64

    @functools.partial(
        pl.kernel, mesh=_mesh(),
        compiler_params=pltpu.CompilerParams(use_tc_tiling_on_sc=False),
        out_type=jax.ShapeDtypeStruct((2, n_acc, width), jnp.float32),
        scratch_types=[
            pltpu.VMEM((128,), jnp.int32),
            pltpu.VMEM((128,), jnp.int32),
            pltpu.VMEM((128, width), jnp.float32),
            pltpu.VMEM((64, width), jnp.float32),
            pltpu.VMEM_SHARED((n_acc, width), jnp.float32),
            pltpu.SemaphoreType.DMA,
        ])
    def k(msg_h, src_h, dst_h, out_h, src_v, dst_v, rows_v, stage_v, acc_sh, sem):
        cid = lax.axis_index("c")
        sid = lax.axis_index("s")
        wid = cid * 16 + sid
        r0 = sid * rpt

        def zrow(r, c):
            for j in range(width // 16):
                stage_v[r, pl.ds(j * 16, 16)] = jnp.zeros((16,), jnp.float32)
            return c

        lax.fori_loop(0, 64, zrow, 0)

        def zchunk(cc, c):
            pltpu.sync_copy(stage_v, acc_sh.at[pl.ds(r0 + cc * 64, 64)])
            return c

        lax.fori_loop(0, chunks, zchunk, 0)
        plsc.subcore_barrier()

        def body(b, c):
            base = (wid * blocks + b) * 128
            pltpu.sync_copy(src_h.at[pl.ds(base, 128)], src_v)
            pltpu.sync_copy(dst_h.at[pl.ds(base, 128)], dst_v)
            pltpu.async_copy(msg_h.at[src_v], rows_v, sem).wait()
            pltpu.sync_copy(rows_v, acc_sh.at[dst_v], add=True)
            return c

        lax.fori_loop(0, blocks, body, 0)
        plsc.subcore_barrier()

        def wchunk(cc, c):
            row = r0 + cc * 64
            pltpu.sync_copy(acc_sh.at[pl.ds(row, 64)], stage_v)
            pltpu.sync_copy(stage_v, out_h.at[cid, pl.ds(row, 64)])
            return c

        lax.fori_loop(0, chunks, wchunk, 0)

    return k(msg, src, dst)


def _tc_b(t, M1):
    R = 1024

    def k(t_ref, m_ref, o_ref):
        y = jnp.dot(t_ref[...], m_ref[...], preferred_element_type=jnp.float32)
        col = lax.broadcasted_iota(jnp.int32, y.shape, 1)
        o_ref[...] = jnp.where(col == 6, 1.0, y)

    return pl.pallas_call(
        k, grid=(NID_PAD // R,),
        in_specs=[pl.BlockSpec((R, D), lambda i: (i, 0)),
                  pl.BlockSpec((D, 16), lambda i: (0, 0))],
        out_specs=pl.BlockSpec((R, 16), lambda i: (i, 0)),
        out_shape=jax.ShapeDtypeStruct((NID_PAD, 16), jnp.float32))(t, M1)


def _tc_d(acc1, T1, B1):
    R = 1024

    def k(a_ref, t_ref, b_ref, o_ref):
        a = a_ref[0] + a_ref[1]
        cnt = jnp.clip(a[:, 6:7], 1.0)
        h = t_ref[:, 8:14] + a[:, 0:6] / cnt + b_ref[0:1, 0:6]
        n = jnp.sqrt(jnp.sum(h * h, axis=1, keepdims=True))
        h = jnp.maximum(h / jnp.clip(n, 1e-12), 0.0)
        o_ref[...] = jnp.concatenate(
            [h, jnp.zeros((R, 10), jnp.float32)], axis=1)

    return pl.pallas_call(
        k, grid=(ACC0 // R,),
        in_specs=[pl.BlockSpec((2, R, 16), lambda i: (0, i, 0)),
                  pl.BlockSpec((R, 16), lambda i: (i, 0)),
                  pl.BlockSpec((8, 16), lambda i: (0, 0))],
        out_specs=pl.BlockSpec((R, 16), lambda i: (i, 0)),
        out_shape=jax.ShapeDtypeStruct((ACC0, 16), jnp.float32))(acc1, T1, B1)


def _tc_f(acc2, acc1, T2, M2A, M2B, B2):
    R = 1024

    def k(a2_ref, a1_ref, t2_ref, ma_ref, mb_ref, b_ref, o_ref):
        a1 = a1_ref[0] + a1_ref[1]
        cnt = jnp.clip(a1[:, 6:7], 1.0)
        a2 = (a2_ref[0] + a2_ref[1]) / cnt
        h = (jnp.dot(t2_ref[...], ma_ref[...], preferred_element_type=jnp.float32)
             + jnp.dot(a2, mb_ref[...], preferred_element_type=jnp.float32)
             + b_ref[0:1, :])
        n = jnp.sqrt(jnp.sum(h * h, axis=1, keepdims=True))
        h = jnp.maximum(h / jnp.clip(n, 1e-12), 0.0)
        col = lax.broadcasted_iota(jnp.int32, h.shape, 1)
        o_ref[...] = jnp.where(col == 36, 1.0, h)

    return pl.pallas_call(
        k, grid=(ACC0 // R,),
        in_specs=[pl.BlockSpec((2, R, 16), lambda i: (0, i, 0)),
                  pl.BlockSpec((2, R, 16), lambda i: (0, i, 0)),
                  pl.BlockSpec((R, 16), lambda i: (i, 0)),
                  pl.BlockSpec((16, 48), lambda i: (0, 0)),
                  pl.BlockSpec((16, 48), lambda i: (0, 0)),
                  pl.BlockSpec((8, 48), lambda i: (0, 0))],
        out_specs=pl.BlockSpec((R, 48), lambda i: (i, 0)),
        out_shape=jax.ShapeDtypeStruct((ACC0, 48), jnp.float32))(
            acc2, acc1, T2, M2A, M2B, B2)


def _tc_h(acc3, M3, B3, M4):
    R = 512

    def k(a_ref, m3_ref, b3_ref, m4_ref, o_ref):
        a = a_ref[0] + a_ref[1]
        a = a / jnp.clip(a[:, 36:37], 1.0)
        h3 = jnp.maximum(
            jnp.dot(a, m3_ref[...], preferred_element_type=jnp.float32)
            + b3_ref[0:1, :], 0.0)
        g = jnp.dot(h3, m4_ref[...], preferred_element_type=jnp.float32)
        col = lax.broadcasted_iota(jnp.int32, g.shape, 1)
        o_ref[...] = jnp.where(col == 128, 1.0, g)

    return pl.pallas_call(
        k, grid=(ACC1 // R,),
        in_specs=[pl.BlockSpec((2, R, 48), lambda i: (0, i, 0)),
                  pl.BlockSpec((48, 224), lambda i: (0, 0)),
                  pl.BlockSpec((8, 224), lambda i: (0, 0)),
                  pl.BlockSpec((224, 144), lambda i: (0, 0))],
        out_specs=pl.BlockSpec((R, 144), lambda i: (i, 0)),
        out_shape=jax.ShapeDtypeStruct((ACC1, 144), jnp.float32))(
            acc3, M3, B3, M4)


def _tc_j(acc4, B4):
    def k(a_ref, b_ref, o_ref):
        a = a_ref[0] + a_ref[1]
        cnt = jnp.clip(a[:, 128:129], 1.0)
        o = a[:, 0:128] / cnt + b_ref[0:1, :]
        o_ref[...] = o[0:N2][None]

    return pl.pallas_call(
        k, grid=(1,),
        in_specs=[pl.BlockSpec((2, ACC2, 144), lambda i: (0, 0, 0)),
                  pl.BlockSpec((8, 128), lambda i: (0, 0))],
        out_specs=pl.BlockSpec((1, N2, 128), lambda i: (0, 0, 0)),
        out_shape=jax.ShapeDtypeStruct((1, N2, 128), jnp.float32))(acc4, B4)


def kernel(x, n_id, edge_index0, edge_index1, edge_index2, res_n_id1,
           res_n_id2, W1, b1, W2, b2, W3, b3, W4, b4):
    i32 = jnp.int32
    f32 = jnp.float32
    x2 = x.reshape(x.shape[1], x.shape[2])

    nid_p = jnp.concatenate(
        [n_id.astype(i32), jnp.zeros((NID_PAD - N0,), i32)])

    def pad_e(ei, ep, dummy):
        e = ei.shape[1]
        s = jnp.concatenate([ei[0].astype(i32), jnp.zeros((ep - e,), i32)])
        d = jnp.concatenate([ei[1].astype(i32), jnp.full((ep - e,), dummy, i32)])
        return s, d

    e0s, e0d = pad_e(edge_index0, E0P, N0)
    e1s, e1d = pad_e(edge_index1, E1P, N1)
    e2s, e2d = pad_e(edge_index2, E2P, N2)

    M1 = jnp.zeros((D, 16), f32).at[:, 0:6].set(W1[D:]).at[:, 8:14].set(W1[:D])
    B1 = jnp.zeros((8, 16), f32).at[0, 0:6].set(b1)
    M2A = jnp.zeros((16, 48), f32).at[0:6, 0:36].set(W2[:6])
    M2B = jnp.zeros((16, 48), f32).at[0:6, 0:36].set(W2[6:])
    B2 = jnp.zeros((8, 48), f32).at[0, 0:36].set(b2)
    M3 = jnp.zeros((48, 224), f32).at[0:36, 0:216].set(W3)
    B3 = jnp.zeros((8, 224), f32).at[0, 0:216].set(b3)
    M4 = jnp.zeros((224, 144), f32).at[0:216, 0:128].set(W4)
    B4 = jnp.zeros((8, 128), f32).at[0].set(b4)

    t = _sc_gather(x2, nid_p)
    T1 = _tc_b(t, M1)
    acc1 = _sc_segsum(T1, e0s, e0d, ACC0, 16, E0P)
    T2 = _tc_d(acc1, T1, B1)
    acc2 = _sc_segsum(T2, e0s, e0d, ACC0, 16, E0P)
    T3 = _tc_f(acc2, acc1, T2, M2A, M2B, B2)
    acc3 = _sc_segsum(T3, e1s, e1d, ACC1, 48, E1P)
    T4 = _tc_h(acc3, M3, B3, M4)
    acc4 = _sc_segsum(T4, e2s, e2d, ACC2, 144, E2P)
    return _tc_j(acc4, B4)


# speedup vs baseline: 8.4943x; 1.4521x over previous
import functools

import jax
import jax.numpy as jnp
from jax import lax
from jax.experimental import pallas as pl
from jax.experimental.pallas import tpu as pltpu
from jax.experimental.pallas import tpu_sc as plsc

N0, N1, N2 = 30000, 8000, 2000
D = 128
NID_PAD = 32768
E0P, E1P, E2P = 491520, 131072, 32768
ACC0, ACC1, ACC2 = 30720, 8192, 2048
NW = 32


def _mesh():
    return plsc.VectorSubcoreMesh(core_axis_name="c", subcore_axis_name="s")


def _sc_gather(x2, nid2):
    blocks = NID_PAD // (NW * 128)

    @functools.partial(
        pl.kernel, mesh=_mesh(),
        out_type=jax.ShapeDtypeStruct((NID_PAD, D), jnp.float32),
        scratch_types=[
            pltpu.VMEM((blocks, 128), jnp.int32),
            pltpu.VMEM((2, 128, D), jnp.float32),
            pltpu.SemaphoreType.DMA,
            pltpu.SemaphoreType.DMA,
        ])
    def k(x_h, nid_h, t_h, idx_v, rows_v, gsem, ssem):
        wid = lax.axis_index("c") * 16 + lax.axis_index("s")
        pltpu.sync_copy(nid_h.at[pl.ds(wid * blocks, blocks)], idx_v)
        pltpu.async_copy(x_h.at[idx_v.at[0]], rows_v.at[0], gsem)

        def body(b, c):
            s = lax.rem(b, 2)
            pltpu.make_async_copy(x_h.at[idx_v.at[b]], rows_v.at[s], gsem).wait()

            @pl.when(b > 0)
            def _():
                pltpu.make_async_copy(
                    rows_v.at[1 - s],
                    t_h.at[pl.ds((wid * blocks + b - 1) * 128, 128)],
                    ssem).wait()

            @pl.when(b < blocks - 1)
            def _():
                pltpu.async_copy(x_h.at[idx_v.at[b + 1]], rows_v.at[1 - s], gsem)

            pltpu.async_copy(
                rows_v.at[s], t_h.at[pl.ds((wid * blocks + b) * 128, 128)], ssem)
            return c

        lax.fori_loop(0, blocks, body, 0)
        pltpu.make_async_copy(
            rows_v.at[(blocks - 1) % 2],
            t_h.at[pl.ds((wid * blocks + blocks - 1) * 128, 128)], ssem).wait()

    return k(x2, nid2)


def _sc_segsum(msg, src2, dst2, zeros, n_acc, width, ep, nbuf):
    blocks = ep // (NW * 128)
    groups = blocks // nbuf
    rpt = n_acc // 16

    @functools.partial(
        pl.kernel, mesh=_mesh(),
        compiler_params=pltpu.CompilerParams(use_tc_tiling_on_sc=False),
        out_type=jax.ShapeDtypeStruct((2, n_acc, width), jnp.float32),
        scratch_types=[
            pltpu.VMEM((blocks, 128), jnp.int32),
            pltpu.VMEM((blocks, 128), jnp.int32),
            pltpu.VMEM((2, nbuf, 128, width), jnp.float32),
            pltpu.VMEM_SHARED((n_acc, width), jnp.float32),
            pltpu.SemaphoreType.DMA,
            pltpu.SemaphoreType.DMA,
        ])
    def k(msg_h, src_h, dst_h, zero_h, out_h, src_i, dst_i, rows_v, acc_sh,
          gsem, ssem):
        cid = lax.axis_index("c")
        sid = lax.axis_index("s")
        wid = cid * 16 + sid
        r0 = sid * rpt
        pltpu.sync_copy(src_h.at[pl.ds(wid * blocks, blocks)], src_i)
        pltpu.sync_copy(dst_h.at[pl.ds(wid * blocks, blocks)], dst_i)
        pltpu.sync_copy(zero_h.at[pl.ds(r0, rpt)], acc_sh.at[pl.ds(r0, rpt)])
        plsc.subcore_barrier()

        for j in range(nbuf):
            pltpu.async_copy(msg_h.at[src_i.at[j]], rows_v.at[0, j], gsem)

        def giter(g, c):
            s = lax.rem(g, 2)
            base = g * nbuf
            for j in range(nbuf):
                pltpu.make_async_copy(
                    msg_h.at[src_i.at[base + j]], rows_v.at[s, j], gsem).wait()

            @pl.when(g > 0)
            def _():
                for j in range(nbuf):
                    pltpu.make_async_copy(
                        rows_v.at[1 - s, j],
                        acc_sh.at[dst_i.at[base - nbuf + j]], ssem).wait()

            @pl.when(g < groups - 1)
            def _():
                for j in range(nbuf):
                    pltpu.async_copy(
                        msg_h.at[src_i.at[base + nbuf + j]],
                        rows_v.at[1 - s, j], gsem)

            for j in range(nbuf):
                pltpu.async_copy(
                    rows_v.at[s, j], acc_sh.at[dst_i.at[base + j]], ssem,
                    add=True)
            return c

        lax.fori_loop(0, groups, giter, 0)
        sl = (groups - 1) % 2
        for j in range(nbuf):
            pltpu.make_async_copy(
                rows_v.at[sl, j],
                acc_sh.at[dst_i.at[(groups - 1) * nbuf + j]], ssem).wait()

        plsc.subcore_barrier()
        pltpu.sync_copy(acc_sh.at[pl.ds(r0, rpt)],
                        out_h.at[cid, pl.ds(r0, rpt)])

    return k(msg, src2, dst2, zeros)


def _tc_b(t, M1):
    R = 1024

    def k(t_ref, m_ref, o_ref):
        y = jnp.dot(t_ref[...], m_ref[...], preferred_element_type=jnp.float32)
        col = lax.broadcasted_iota(jnp.int32, y.shape, 1)
        o_ref[...] = jnp.where(col == 6, 1.0, y)

    return pl.pallas_call(
        k, grid=(NID_PAD // R,),
        in_specs=[pl.BlockSpec((R, D), lambda i: (i, 0)),
                  pl.BlockSpec((D, 16), lambda i: (0, 0))],
        out_specs=pl.BlockSpec((R, 16), lambda i: (i, 0)),
        out_shape=jax.ShapeDtypeStruct((NID_PAD, 16), jnp.float32))(t, M1)


def _tc_d(acc1, T1, B1):
    R = 1024

    def k(a_ref, t_ref, b_ref, o_ref):
        a = a_ref[0] + a_ref[1]
        cnt = jnp.clip(a[:, 6:7], 1.0)
        h = t_ref[:, 8:14] + a[:, 0:6] / cnt + b_ref[0:1, 0:6]
        n = jnp.sqrt(jnp.sum(h * h, axis=1, keepdims=True))
        h = jnp.maximum(h / jnp.clip(n, 1e-12), 0.0)
        o_ref[...] = jnp.concatenate(
            [h, jnp.zeros((R, 10), jnp.float32)], axis=1)

    return pl.pallas_call(
        k, grid=(ACC0 // R,),
        in_specs=[pl.BlockSpec((2, R, 16), lambda i: (0, i, 0)),
                  pl.BlockSpec((R, 16), lambda i: (i, 0)),
                  pl.BlockSpec((8, 16), lambda i: (0, 0))],
        out_specs=pl.BlockSpec((R, 16), lambda i: (i, 0)),
        out_shape=jax.ShapeDtypeStruct((ACC0, 16), jnp.float32))(acc1, T1, B1)


def _tc_f(acc2, acc1, T2, M2A, M2B, B2):
    R = 1024

    def k(a2_ref, a1_ref, t2_ref, ma_ref, mb_ref, b_ref, o_ref):
        a1 = a1_ref[0] + a1_ref[1]
        cnt = jnp.clip(a1[:, 6:7], 1.0)
        a2 = (a2_ref[0] + a2_ref[1]) / cnt
        h = (jnp.dot(t2_ref[...], ma_ref[...], preferred_element_type=jnp.float32)
             + jnp.dot(a2, mb_ref[...], preferred_element_type=jnp.float32)
             + b_ref[0:1, :])
        n = jnp.sqrt(jnp.sum(h * h, axis=1, keepdims=True))
        h = jnp.maximum(h / jnp.clip(n, 1e-12), 0.0)
        col = lax.broadcasted_iota(jnp.int32, h.shape, 1)
        o_ref[...] = jnp.where(col == 36, 1.0, h)

    return pl.pallas_call(
        k, grid=(ACC0 // R,),
        in_specs=[pl.BlockSpec((2, R, 16), lambda i: (0, i, 0)),
                  pl.BlockSpec((2, R, 16), lambda i: (0, i, 0)),
                  pl.BlockSpec((R, 16), lambda i: (i, 0)),
                  pl.BlockSpec((16, 48), lambda i: (0, 0)),
                  pl.BlockSpec((16, 48), lambda i: (0, 0)),
                  pl.BlockSpec((8, 48), lambda i: (0, 0))],
        out_specs=pl.BlockSpec((R, 48), lambda i: (i, 0)),
        out_shape=jax.ShapeDtypeStruct((ACC0, 48), jnp.float32))(
            acc2, acc1, T2, M2A, M2B, B2)


def _tc_h(acc3, M3, B3, M4):
    R = 512

    def k(a_ref, m3_ref, b3_ref, m4_ref, o_ref):
        a = a_ref[0] + a_ref[1]
        a = a / jnp.clip(a[:, 36:37], 1.0)
        h3 = jnp.maximum(
            jnp.dot(a, m3_ref[...], preferred_element_type=jnp.float32)
            + b3_ref[0:1, :], 0.0)
        g = jnp.dot(h3, m4_ref[...], preferred_element_type=jnp.float32)
        col = lax.broadcasted_iota(jnp.int32, g.shape, 1)
        o_ref[...] = jnp.where(col == 128, 1.0, g)

    return pl.pallas_call(
        k, grid=(ACC1 // R,),
        in_specs=[pl.BlockSpec((2, R, 48), lambda i: (0, i, 0)),
                  pl.BlockSpec((48, 224), lambda i: (0, 0)),
                  pl.BlockSpec((8, 224), lambda i: (0, 0)),
                  pl.BlockSpec((224, 144), lambda i: (0, 0))],
        out_specs=pl.BlockSpec((R, 144), lambda i: (i, 0)),
        out_shape=jax.ShapeDtypeStruct((ACC1, 144), jnp.float32))(
            acc3, M3, B3, M4)


def _tc_j(acc4, B4):
    def k(a_ref, b_ref, o_ref):
        a = a_ref[0] + a_ref[1]
        cnt = jnp.clip(a[:, 128:129], 1.0)
        o = a[:, 0:128] / cnt + b_ref[0:1, :]
        o_ref[...] = o[0:N2][None]

    return pl.pallas_call(
        k, grid=(1,),
        in_specs=[pl.BlockSpec((2, ACC2, 144), lambda i: (0, 0, 0)),
                  pl.BlockSpec((8, 128), lambda i: (0, 0))],
        out_specs=pl.BlockSpec((1, N2, 128), lambda i: (0, 0, 0)),
        out_shape=jax.ShapeDtypeStruct((1, N2, 128), jnp.float32))(acc4, B4)


def kernel(x, n_id, edge_index0, edge_index1, edge_index2, res_n_id1,
           res_n_id2, W1, b1, W2, b2, W3, b3, W4, b4):
    i32 = jnp.int32
    f32 = jnp.float32
    x2 = x.reshape(x.shape[1], x.shape[2])

    nid_p = jnp.concatenate(
        [n_id.astype(i32), jnp.zeros((NID_PAD - N0,), i32)]).reshape(-1, 128)

    def pad_e(ei, ep, dummy):
        e = ei.shape[1]
        s = jnp.concatenate([ei[0].astype(i32), jnp.zeros((ep - e,), i32)])
        d = jnp.concatenate([ei[1].astype(i32), jnp.full((ep - e,), dummy, i32)])
        return s.reshape(-1, 128), d.reshape(-1, 128)

    e0s, e0d = pad_e(edge_index0, E0P, N0)
    e1s, e1d = pad_e(edge_index1, E1P, N1)
    e2s, e2d = pad_e(edge_index2, E2P, N2)
    z0 = jnp.zeros((ACC0, 16), f32)
    z1 = jnp.zeros((ACC1, 48), f32)
    z2 = jnp.zeros((ACC2, 144), f32)

    M1 = jnp.zeros((D, 16), f32).at[:, 0:6].set(W1[D:]).at[:, 8:14].set(W1[:D])
    B1 = jnp.zeros((8, 16), f32).at[0, 0:6].set(b1)
    M2A = jnp.zeros((16, 48), f32).at[0:6, 0:36].set(W2[:6])
    M2B = jnp.zeros((16, 48), f32).at[0:6, 0:36].set(W2[6:])
    B2 = jnp.zeros((8, 48), f32).at[0, 0:36].set(b2)
    M3 = jnp.zeros((48, 224), f32).at[0:36, 0:216].set(W3)
    B3 = jnp.zeros((8, 224), f32).at[0, 0:216].set(b3)
    M4 = jnp.zeros((224, 144), f32).at[0:216, 0:128].set(W4)
    B4 = jnp.zeros((8, 128), f32).at[0].set(b4)

    t = _sc_gather(x2, nid_p)
    T1 = _tc_b(t, M1)
    acc1 = _sc_segsum(T1, e0s, e0d, z0, ACC0, 16, E0P, 6)
    T2 = _tc_d(acc1, T1, B1)
    acc2 = _sc_segsum(T2, e0s, e0d, z0, ACC0, 16, E0P, 6)
    T3 = _tc_f(acc2, acc1, T2, M2A, M2B, B2)
    acc3 = _sc_segsum(T3, e1s, e1d, z1, ACC1, 48, E1P, 4)
    T4 = _tc_h(acc3, M3, B3, M4)
    acc4 = _sc_segsum(T4, e2s, e2d, z2, ACC2, 144, E2P, 2)
    return _tc_j(acc4, B4)


# speedup vs baseline: 8.8028x; 1.0363x over previous
import functools

import jax
import jax.numpy as jnp
from jax import lax
from jax.experimental import pallas as pl
from jax.experimental.pallas import tpu as pltpu
from jax.experimental.pallas import tpu_sc as plsc

N0, N1, N2 = 30000, 8000, 2000
D = 128
NID_PAD = 32768
E0P, E1P, E2P = 491520, 131072, 32768
ACC0, ACC1, ACC2 = 30720, 8192, 2048
NW = 32


def _mesh():
    return plsc.VectorSubcoreMesh(core_axis_name="c", subcore_axis_name="s")


def _sc_gather(y, nid2):
    blocks = NID_PAD // (NW * 128)

    @functools.partial(
        pl.kernel, mesh=_mesh(),
        compiler_params=pltpu.CompilerParams(use_tc_tiling_on_sc=False),
        out_type=jax.ShapeDtypeStruct((NID_PAD, 16), jnp.float32),
        scratch_types=[
            pltpu.VMEM((blocks, 128), jnp.int32),
            pltpu.VMEM((2, 128, 16), jnp.float32),
            pltpu.SemaphoreType.DMA,
            pltpu.SemaphoreType.DMA,
        ])
    def k(y_h, nid_h, t_h, idx_v, rows_v, gsem, ssem):
        wid = lax.axis_index("c") * 16 + lax.axis_index("s")
        pltpu.sync_copy(nid_h.at[pl.ds(wid * blocks, blocks)], idx_v)
        pltpu.async_copy(y_h.at[idx_v.at[0]], rows_v.at[0], gsem)

        def body(b, c):
            s = lax.rem(b, 2)
            pltpu.make_async_copy(y_h.at[idx_v.at[b]], rows_v.at[s], gsem).wait()

            @pl.when(b > 0)
            def _():
                pltpu.make_async_copy(
                    rows_v.at[1 - s],
                    t_h.at[pl.ds((wid * blocks + b - 1) * 128, 128)],
                    ssem).wait()

            @pl.when(b < blocks - 1)
            def _():
                pltpu.async_copy(y_h.at[idx_v.at[b + 1]], rows_v.at[1 - s], gsem)

            pltpu.async_copy(
                rows_v.at[s], t_h.at[pl.ds((wid * blocks + b) * 128, 128)], ssem)
            return c

        lax.fori_loop(0, blocks, body, 0)
        pltpu.make_async_copy(
            rows_v.at[(blocks - 1) % 2],
            t_h.at[pl.ds((wid * blocks + blocks - 1) * 128, 128)], ssem).wait()

    return k(y, nid2)


def _sc_segsum(msg, src2, dst2, zeros, n_acc, width, ep, nbuf):
    blocks = ep // (NW * 128)
    groups = blocks // nbuf
    rpt = n_acc // 16

    @functools.partial(
        pl.kernel, mesh=_mesh(),
        compiler_params=pltpu.CompilerParams(use_tc_tiling_on_sc=False),
        out_type=jax.ShapeDtypeStruct((2, n_acc, width), jnp.float32),
        scratch_types=[
            pltpu.VMEM((blocks, 128), jnp.int32),
            pltpu.VMEM((blocks, 128), jnp.int32),
            pltpu.VMEM((2, nbuf, 128, width), jnp.float32),
            pltpu.VMEM_SHARED((n_acc, width), jnp.float32),
            pltpu.SemaphoreType.DMA,
            pltpu.SemaphoreType.DMA,
        ])
    def k(msg_h, src_h, dst_h, zero_h, out_h, src_i, dst_i, rows_v, acc_sh,
          gsem, ssem):
        cid = lax.axis_index("c")
        sid = lax.axis_index("s")
        wid = cid * 16 + sid
        r0 = sid * rpt
        pltpu.sync_copy(src_h.at[pl.ds(wid * blocks, blocks)], src_i)
        pltpu.sync_copy(dst_h.at[pl.ds(wid * blocks, blocks)], dst_i)
        pltpu.sync_copy(zero_h.at[pl.ds(r0, rpt)], acc_sh.at[pl.ds(r0, rpt)])
        plsc.subcore_barrier()

        for j in range(nbuf):
            pltpu.async_copy(msg_h.at[src_i.at[j]], rows_v.at[0, j], gsem)

        def giter(g, c):
            s = lax.rem(g, 2)
            base = g * nbuf
            for j in range(nbuf):
                pltpu.make_async_copy(
                    msg_h.at[src_i.at[base + j]], rows_v.at[s, j], gsem).wait()

            @pl.when(g > 0)
            def _():
                for j in range(nbuf):
                    pltpu.make_async_copy(
                        rows_v.at[1 - s, j],
                        acc_sh.at[dst_i.at[base - nbuf + j]], ssem).wait()

            @pl.when(g < groups - 1)
            def _():
                for j in range(nbuf):
                    pltpu.async_copy(
                        msg_h.at[src_i.at[base + nbuf + j]],
                        rows_v.at[1 - s, j], gsem)

            for j in range(nbuf):
                pltpu.async_copy(
                    rows_v.at[s, j], acc_sh.at[dst_i.at[base + j]], ssem,
                    add=True)
            return c

        lax.fori_loop(0, groups, giter, 0)
        sl = (groups - 1) % 2
        for j in range(nbuf):
            pltpu.make_async_copy(
                rows_v.at[sl, j],
                acc_sh.at[dst_i.at[(groups - 1) * nbuf + j]], ssem).wait()

        plsc.subcore_barrier()
        pltpu.sync_copy(acc_sh.at[pl.ds(r0, rpt)],
                        out_h.at[cid, pl.ds(r0, rpt)])

    return k(msg, src2, dst2, zeros)


def _tc_proj(x2, M1):
    R = 2000

    def k(x_ref, m_ref, o_ref):
        y = jnp.dot(x_ref[...], m_ref[...], preferred_element_type=jnp.float32)
        col = lax.broadcasted_iota(jnp.int32, y.shape, 1)
        o_ref[...] = jnp.where(col == 6, 1.0, y)

    n = x2.shape[0]
    return pl.pallas_call(
        k, grid=(n // R,),
        in_specs=[pl.BlockSpec((R, D), lambda i: (i, 0)),
                  pl.BlockSpec((D, 16), lambda i: (0, 0))],
        out_specs=pl.BlockSpec((R, 16), lambda i: (i, 0)),
        out_shape=jax.ShapeDtypeStruct((n, 16), jnp.float32))(x2, M1)


def _tc_d(acc1, T1, B1):
    R = 1024

    def k(a_ref, t_ref, b_ref, o_ref):
        a = a_ref[0] + a_ref[1]
        cnt = jnp.clip(a[:, 6:7], 1.0)
        h = t_ref[:, 8:14] + a[:, 0:6] / cnt + b_ref[0:1, 0:6]
        n = jnp.sqrt(jnp.sum(h * h, axis=1, keepdims=True))
        h = jnp.maximum(h / jnp.clip(n, 1e-12), 0.0)
        o_ref[...] = jnp.concatenate(
            [h, jnp.zeros((R, 10), jnp.float32)], axis=1)

    return pl.pallas_call(
        k, grid=(ACC0 // R,),
        in_specs=[pl.BlockSpec((2, R, 16), lambda i: (0, i, 0)),
                  pl.BlockSpec((R, 16), lambda i: (i, 0)),
                  pl.BlockSpec((8, 16), lambda i: (0, 0))],
        out_specs=pl.BlockSpec((R, 16), lambda i: (i, 0)),
        out_shape=jax.ShapeDtypeStruct((ACC0, 16), jnp.float32))(acc1, T1, B1)


def _tc_f(acc2, acc1, T2, M2A, M2B, B2):
    R = 1024

    def k(a2_ref, a1_ref, t2_ref, ma_ref, mb_ref, b_ref, o_ref):
        a1 = a1_ref[0] + a1_ref[1]
        cnt = jnp.clip(a1[:, 6:7], 1.0)
        a2 = (a2_ref[0] + a2_ref[1]) / cnt
        h = (jnp.dot(t2_ref[...], ma_ref[...], preferred_element_type=jnp.float32)
             + jnp.dot(a2, mb_ref[...], preferred_element_type=jnp.float32)
             + b_ref[0:1, :])
        n = jnp.sqrt(jnp.sum(h * h, axis=1, keepdims=True))
        h = jnp.maximum(h / jnp.clip(n, 1e-12), 0.0)
        col = lax.broadcasted_iota(jnp.int32, h.shape, 1)
        o_ref[...] = jnp.where(col == 36, 1.0, h)

    return pl.pallas_call(
        k, grid=(ACC0 // R,),
        in_specs=[pl.BlockSpec((2, R, 16), lambda i: (0, i, 0)),
                  pl.BlockSpec((2, R, 16), lambda i: (0, i, 0)),
                  pl.BlockSpec((R, 16), lambda i: (i, 0)),
                  pl.BlockSpec((16, 48), lambda i: (0, 0)),
                  pl.BlockSpec((16, 48), lambda i: (0, 0)),
                  pl.BlockSpec((8, 48), lambda i: (0, 0))],
        out_specs=pl.BlockSpec((R, 48), lambda i: (i, 0)),
        out_shape=jax.ShapeDtypeStruct((ACC0, 48), jnp.float32))(
            acc2, acc1, T2, M2A, M2B, B2)


def _tc_h(acc3, M3, B3, M4):
    R = 512

    def k(a_ref, m3_ref, b3_ref, m4_ref, o_ref):
        a = a_ref[0] + a_ref[1]
        a = a / jnp.clip(a[:, 36:37], 1.0)
        h3 = jnp.maximum(
            jnp.dot(a, m3_ref[...], preferred_element_type=jnp.float32)
            + b3_ref[0:1, :], 0.0)
        g = jnp.dot(h3, m4_ref[...], preferred_element_type=jnp.float32)
        col = lax.broadcasted_iota(jnp.int32, g.shape, 1)
        o_ref[...] = jnp.where(col == 128, 1.0, g)

    return pl.pallas_call(
        k, grid=(ACC1 // R,),
        in_specs=[pl.BlockSpec((2, R, 48), lambda i: (0, i, 0)),
                  pl.BlockSpec((48, 224), lambda i: (0, 0)),
                  pl.BlockSpec((8, 224), lambda i: (0, 0)),
                  pl.BlockSpec((224, 144), lambda i: (0, 0))],
        out_specs=pl.BlockSpec((R, 144), lambda i: (i, 0)),
        out_shape=jax.ShapeDtypeStruct((ACC1, 144), jnp.float32))(
            acc3, M3, B3, M4)


def _tc_j(acc4, B4):
    def k(a_ref, b_ref, o_ref):
        a = a_ref[0] + a_ref[1]
        cnt = jnp.clip(a[:, 128:129], 1.0)
        o = a[:, 0:128] / cnt + b_ref[0:1, :]
        o_ref[...] = o[0:N2][None]

    return pl.pallas_call(
        k, grid=(1,),
        in_specs=[pl.BlockSpec((2, ACC2, 144), lambda i: (0, 0, 0)),
                  pl.BlockSpec((8, 128), lambda i: (0, 0))],
        out_specs=pl.BlockSpec((1, N2, 128), lambda i: (0, 0, 0)),
        out_shape=jax.ShapeDtypeStruct((1, N2, 128), jnp.float32))(acc4, B4)


def kernel(x, n_id, edge_index0, edge_index1, edge_index2, res_n_id1,
           res_n_id2, W1, b1, W2, b2, W3, b3, W4, b4):
    i32 = jnp.int32
    f32 = jnp.float32
    x2 = x.reshape(x.shape[1], x.shape[2])

    nid_p = jnp.concatenate(
        [n_id.astype(i32), jnp.zeros((NID_PAD - N0,), i32)]).reshape(-1, 128)

    def pad_e(ei, ep, dummy):
        e = ei.shape[1]
        s = jnp.concatenate([ei[0].astype(i32), jnp.zeros((ep - e,), i32)])
        d = jnp.concatenate([ei[1].astype(i32), jnp.full((ep - e,), dummy, i32)])
        return s.reshape(-1, 128), d.reshape(-1, 128)

    e0s, e0d = pad_e(edge_index0, E0P, N0)
    e1s, e1d = pad_e(edge_index1, E1P, N1)
    e2s, e2d = pad_e(edge_index2, E2P, N2)
    z0 = jnp.zeros((ACC0, 16), f32)
    z1 = jnp.zeros((ACC1, 48), f32)
    z2 = jnp.zeros((ACC2, 144), f32)

    M1 = jnp.zeros((D, 16), f32).at[:, 0:6].set(W1[D:]).at[:, 8:14].set(W1[:D])
    B1 = jnp.zeros((8, 16), f32).at[0, 0:6].set(b1)
    M2A = jnp.zeros((16, 48), f32).at[0:6, 0:36].set(W2[:6])
    M2B = jnp.zeros((16, 48), f32).at[0:6, 0:36].set(W2[6:])
    B2 = jnp.zeros((8, 48), f32).at[0, 0:36].set(b2)
    M3 = jnp.zeros((48, 224), f32).at[0:36, 0:216].set(W3)
    B3 = jnp.zeros((8, 224), f32).at[0, 0:216].set(b3)
    M4 = jnp.zeros((224, 144), f32).at[0:216, 0:128].set(W4)
    B4 = jnp.zeros((8, 128), f32).at[0].set(b4)

    Y = _tc_proj(x2, M1)
    T1 = _sc_gather(Y, nid_p)
    acc1 = _sc_segsum(T1, e0s, e0d, z0, ACC0, 16, E0P, 6)
    T2 = _tc_d(acc1, T1, B1)
    acc2 = _sc_segsum(T2, e0s, e0d, z0, ACC0, 16, E0P, 6)
    T3 = _tc_f(acc2, acc1, T2, M2A, M2B, B2)
    acc3 = _sc_segsum(T3, e1s, e1d, z1, ACC1, 48, E1P, 4)
    T4 = _tc_h(acc3, M3, B3, M4)
    acc4 = _sc_segsum(T4, e2s, e2d, z2, ACC2, 144, E2P, 2)
    return _tc_j(acc4, B4)
